# Initial kernel scaffold; baseline (speedup 1.0000x reference)
import jax
import jax.numpy as jnp
from jax.experimental import pallas as pl


def kernel(points, bev_volume, W1, b1, W2, b2):
    raise NotImplementedError("write your pallas kernel here")


# speedup vs baseline: 9.5476x; 9.5476x over previous
import functools

import jax
import jax.numpy as jnp
from jax import lax
from jax.experimental import pallas as pl
from jax.experimental.pallas import tpu as pltpu
from jax.experimental.pallas import tpu_sc as plsc

N_POINTS = 20000
N_PAD = 20480
N_KEY = 2048
NSAMPLE = 32
RADIUS2 = 0.8 * 0.8
C_BEV = 256
H = 176
W = 200
BOUND = 4.0
PADV = 1.0e6
INVALID = 1.0e10


def _fps_body(x_ref, y_ref, z_ref, d0_ref, out_ref, dist_ref):
    dist_ref[...] = d0_ref[...]
    rows = lax.broadcasted_iota(jnp.int32, (8, 2560), 0)
    cols = lax.broadcasted_iota(jnp.int32, (8, 2560), 1)
    lin = rows * 2560 + cols
    lanes = lax.broadcasted_iota(jnp.int32, (1, 128), 1)

    lx0 = x_ref[0:1, 0:1]
    ly0 = y_ref[0:1, 0:1]
    lz0 = z_ref[0:1, 0:1]
    out_ref[0:1, :] = jnp.where(
        lanes == 0, lx0, jnp.where(lanes == 1, ly0, jnp.where(lanes == 2, lz0, 0.0)))

    def body(i, carry):
        lx, ly, lz = carry
        dx = x_ref[...] - lx
        dy = y_ref[...] - ly
        dz = z_ref[...] - lz
        d = dx * dx + dy * dy + dz * dz
        dn = jnp.minimum(dist_ref[...], d)
        dist_ref[...] = dn
        m = jnp.max(dn)
        sel = jnp.min(jnp.where(dn == m, lin, jnp.int32(2 ** 30)))
        hot = lin == sel
        nlx = jnp.sum(jnp.where(hot, x_ref[...], 0.0), keepdims=True)[0:1, 0:1]
        nly = jnp.sum(jnp.where(hot, y_ref[...], 0.0), keepdims=True)[0:1, 0:1]
        nlz = jnp.sum(jnp.where(hot, z_ref[...], 0.0), keepdims=True)[0:1, 0:1]
        row = jnp.where(
            lanes == 0, nlx, jnp.where(lanes == 1, nly, jnp.where(lanes == 2, nlz, 0.0)))
        out_ref[pl.ds(i, 1), :] = row
        return (nlx, nly, nlz)

    lax.fori_loop(1, N_KEY, body, (lx0, ly0, lz0))


def _fps(x8, y8, z8, d0):
    return pl.pallas_call(
        _fps_body,
        out_shape=jax.ShapeDtypeStruct((N_KEY, 128), jnp.float32),
        scratch_shapes=[pltpu.VMEM((8, 2560), jnp.float32)],
    )(x8, y8, z8, d0)


KB = 8


def _ballq_body(key_ref, x_ref, y_ref, z_ref, out_ref, m_ref):
    kx = key_ref[:, 0:1]
    ky = key_ref[:, 1:2]
    kz = key_ref[:, 2:3]
    dx = kx - x_ref[...]
    dy = ky - y_ref[...]
    dz = kz - z_ref[...]
    d2 = dx * dx + dy * dy + dz * dz
    m_ref[...] = jnp.where(d2 <= RADIUS2, d2, INVALID)

    lanelin = lax.broadcasted_iota(jnp.int32, (KB, N_PAD), 1)
    out_ref[...] = jnp.zeros((KB, 128), jnp.int32)

    first_sel = None
    for s in range(NSAMPLE):
        cur = m_ref[...]
        m = jnp.min(cur, axis=1, keepdims=True)
        sel = jnp.min(jnp.where(cur == m, lanelin, jnp.int32(2 ** 30)),
                      axis=1, keepdims=True)
        m_ref[...] = jnp.where(lanelin == sel, INVALID, cur)
        if s == 0:
            first_sel = sel
            rec = sel
        else:
            rec = jnp.where(m < 1.0e9, sel, first_sel)
        out_ref[:, s:s + 1] = rec


def _ballq(keyrows, x1, y1, z1):
    return pl.pallas_call(
        _ballq_body,
        grid=(N_KEY // KB,),
        in_specs=[
            pl.BlockSpec((KB, 128), lambda i: (i, 0)),
            pl.BlockSpec((1, N_PAD), lambda i: (0, 0)),
            pl.BlockSpec((1, N_PAD), lambda i: (0, 0)),
            pl.BlockSpec((1, N_PAD), lambda i: (0, 0)),
        ],
        out_specs=pl.BlockSpec((KB, 128), lambda i: (i, 0)),
        out_shape=jax.ShapeDtypeStruct((N_KEY, 128), jnp.int32),
        scratch_shapes=[pltpu.VMEM((KB, N_PAD), jnp.float32)],
    )(keyrows, x1, y1, z1)


NW = 32
NBR_TOTAL = N_KEY * NSAMPLE
NBR_PER_W = NBR_TOTAL // NW
NBR_CHUNKS = NBR_PER_W // 128


def _sc_nbr_gather(points128, idx3d):
    mesh = plsc.VectorSubcoreMesh(core_axis_name="c", subcore_axis_name="s")

    @functools.partial(
        pl.kernel, mesh=mesh,
        out_type=jax.ShapeDtypeStruct((NBR_TOTAL, 128), jnp.float32),
        scratch_types=[
            pltpu.VMEM((NBR_CHUNKS, 128), jnp.int32),
            pltpu.VMEM((128, 128), jnp.float32),
            pltpu.VMEM((128, 128), jnp.float32),
            pltpu.SemaphoreType.DMA,
            pltpu.SemaphoreType.DMA,
        ],
    )
    def k(table_hbm, idx_hbm, out_hbm, idx_v, buf0, buf1, sem0, sem1):
        wid = lax.axis_index("s") * 2 + lax.axis_index("c")
        base = wid * NBR_PER_W
        pltpu.sync_copy(idx_hbm.at[wid], idx_v)
        bufs = (buf0, buf1)
        sems = (sem0, sem1)
        copies = [None, None]
        for j in range(NBR_CHUNKS):
            p = j % 2
            if copies[p] is not None:
                copies[p].wait()
                pltpu.sync_copy(bufs[p],
                                out_hbm.at[pl.ds(base + (j - 2) * 128, 128)])
            copies[p] = pltpu.async_copy(
                table_hbm.at[idx_v.at[j]], bufs[p], sems[p])
        for j in (NBR_CHUNKS - 2, NBR_CHUNKS - 1):
            p = j % 2
            copies[p].wait()
            pltpu.sync_copy(bufs[p], out_hbm.at[pl.ds(base + j * 128, 128)])

    return k(points128, idx3d)


KEY_PER_W = N_KEY // NW


def _sc_bev_gather(table, kx, ky):
    mesh = plsc.VectorSubcoreMesh(core_axis_name="c", subcore_axis_name="s")
    out1 = jax.ShapeDtypeStruct((N_KEY, C_BEV), jnp.float32)

    @functools.partial(
        pl.kernel, mesh=mesh,
        out_type=(out1, out1, out1, out1),
        scratch_types=[
            pltpu.VMEM((KEY_PER_W,), jnp.float32),
            pltpu.VMEM((KEY_PER_W,), jnp.float32),
            pltpu.VMEM((KEY_PER_W,), jnp.int32),
            pltpu.VMEM((KEY_PER_W,), jnp.int32),
            pltpu.VMEM((KEY_PER_W,), jnp.int32),
            pltpu.VMEM((KEY_PER_W,), jnp.int32),
            pltpu.VMEM((KEY_PER_W, C_BEV), jnp.float32),
            pltpu.VMEM((KEY_PER_W, C_BEV), jnp.float32),
            pltpu.VMEM((KEY_PER_W, C_BEV), jnp.float32),
            pltpu.VMEM((KEY_PER_W, C_BEV), jnp.float32),
            pltpu.SemaphoreType.DMA,
        ],
    )
    def k(table_hbm, kx_hbm, ky_hbm, o00, o01, o10, o11,
          kx_v, ky_v, i00, i01, i10, i11, f00, f01, f10, f11, sem):
        wid = lax.axis_index("s") * 2 + lax.axis_index("c")
        base = wid * KEY_PER_W
        pltpu.sync_copy(kx_hbm.at[pl.ds(base, KEY_PER_W)], kx_v)
        pltpu.sync_copy(ky_hbm.at[pl.ds(base, KEY_PER_W)], ky_v)
        for j in range(KEY_PER_W // 16):
            sl = pl.ds(j * 16, 16)
            x = kx_v[sl]
            y = ky_v[sl]
            u = (x + BOUND) / (2.0 * BOUND) * (W - 1)
            v = (y + BOUND) / (2.0 * BOUND) * (H - 1)
            u = jnp.minimum(jnp.maximum(u, 0.0), W - 1.0)
            v = jnp.minimum(jnp.maximum(v, 0.0), H - 1.0)
            u0 = u.astype(jnp.int32)
            v0 = v.astype(jnp.int32)
            u1 = jnp.minimum(u0 + 1, W - 1)
            v1 = jnp.minimum(v0 + 1, H - 1)
            i00[sl] = v0 * W + u0
            i01[sl] = v0 * W + u1
            i10[sl] = v1 * W + u0
            i11[sl] = v1 * W + u1
        copies = [
            pltpu.async_copy(table_hbm.at[i00], f00, sem),
            pltpu.async_copy(table_hbm.at[i01], f01, sem),
            pltpu.async_copy(table_hbm.at[i10], f10, sem),
            pltpu.async_copy(table_hbm.at[i11], f11, sem),
        ]
        for c in copies:
            c.wait()
        dst = pl.ds(base, KEY_PER_W)
        pltpu.sync_copy(f00, o00.at[dst])
        pltpu.sync_copy(f01, o01.at[dst])
        pltpu.sync_copy(f10, o10.at[dst])
        pltpu.sync_copy(f11, o11.at[dst])

    return k(table, kx, ky)


MB = 256


def _mlp_body(g_ref, key_ref, w1_ref, b1_ref, w2_ref, b2_ref, out_ref):
    g4 = g_ref[:, 0:4]
    a = jnp.dot(g4, w1_ref[...], preferred_element_type=jnp.float32)
    kxyz = key_ref[:, 0:3]
    b = jnp.dot(kxyz, w1_ref[0:3, :], preferred_element_type=jnp.float32)
    a3 = a.reshape(MB, NSAMPLE, 32)
    h1 = jnp.maximum(a3 - b[:, None, :] + b1_ref[0:1, :][None, :, :], 0.0)
    h2 = jnp.dot(h1.reshape(MB * NSAMPLE, 32), w2_ref[...],
                 preferred_element_type=jnp.float32)
    h2 = jnp.maximum(h2 + b2_ref[0:1, :], 0.0)
    out_ref[...] = jnp.max(h2.reshape(MB, NSAMPLE, 64), axis=1)


def _mlp(gathered, keyrows, W1, b1, W2, b2):
    return pl.pallas_call(
        _mlp_body,
        grid=(N_KEY // MB,),
        in_specs=[
            pl.BlockSpec((MB * NSAMPLE, 128), lambda i: (i, 0)),
            pl.BlockSpec((MB, 128), lambda i: (i, 0)),
            pl.BlockSpec((4, 32), lambda i: (0, 0)),
            pl.BlockSpec((1, 32), lambda i: (0, 0)),
            pl.BlockSpec((32, 64), lambda i: (0, 0)),
            pl.BlockSpec((1, 64), lambda i: (0, 0)),
        ],
        out_specs=pl.BlockSpec((MB, 64), lambda i: (i, 0)),
        out_shape=jax.ShapeDtypeStruct((N_KEY, 64), jnp.float32),
    )(gathered, keyrows, W1, b1, W2, b2)


def _blend_body(pnet_ref, f00_ref, f01_ref, f10_ref, f11_ref, key_ref, out_ref):
    kx = key_ref[:, 0:1]
    ky = key_ref[:, 1:2]
    u = (kx + BOUND) / (2.0 * BOUND) * (W - 1)
    v = (ky + BOUND) / (2.0 * BOUND) * (H - 1)
    u = jnp.clip(u, 0.0, W - 1.0)
    v = jnp.clip(v, 0.0, H - 1.0)
    wu = u - jnp.floor(u)
    wv = v - jnp.floor(v)
    bev = (f00_ref[...] * (1.0 - wu) * (1.0 - wv)
           + f01_ref[...] * wu * (1.0 - wv)
           + f10_ref[...] * (1.0 - wu) * wv
           + f11_ref[...] * wu * wv)
    out_ref[:, 0:64] = pnet_ref[...]
    out_ref[:, 64:320] = bev


def _blend(pnet, f00, f01, f10, f11, keyrows):
    return pl.pallas_call(
        _blend_body,
        out_shape=jax.ShapeDtypeStruct((N_KEY, 320), jnp.float32),
    )(pnet, f00, f01, f10, f11, keyrows)


def kernel(points, bev_volume, W1, b1, W2, b2):
    xyz = points[:, :3]
    pad = jnp.full((N_PAD - N_POINTS,), PADV, jnp.float32)
    xp = jnp.concatenate([xyz[:, 0], pad])
    yp = jnp.concatenate([xyz[:, 1], pad])
    zp = jnp.concatenate([xyz[:, 2], pad])
    d0 = jnp.concatenate([
        jnp.full((N_POINTS,), INVALID, jnp.float32),
        jnp.full((N_PAD - N_POINTS,), -INVALID, jnp.float32),
    ]).reshape(8, 2560)

    keyrows = _fps(xp.reshape(8, 2560), yp.reshape(8, 2560),
                   zp.reshape(8, 2560), d0)

    nbr_pad = _ballq(keyrows, xp.reshape(1, N_PAD), yp.reshape(1, N_PAD),
                     zp.reshape(1, N_PAD))
    idx3d = nbr_pad[:, :NSAMPLE].reshape(NW, NBR_CHUNKS, 128)

    points128 = jnp.pad(points, ((0, 0), (0, 124)))
    gathered = _sc_nbr_gather(points128, idx3d)

    bev_t = jnp.transpose(bev_volume[0], (1, 2, 0)).reshape(H * W, C_BEV)
    kx = keyrows[:, 0]
    ky = keyrows[:, 1]
    f00, f01, f10, f11 = _sc_bev_gather(bev_t, kx, ky)

    pnet = _mlp(gathered, keyrows, W1, b1.reshape(1, 32), W2, b2.reshape(1, 64))

    return _blend(pnet, f00, f01, f10, f11, keyrows)


# speedup vs baseline: 17.5935x; 1.8427x over previous
import functools

import jax
import jax.numpy as jnp
from jax import lax
from jax.experimental import pallas as pl
from jax.experimental.pallas import tpu as pltpu
from jax.experimental.pallas import tpu_sc as plsc

N_POINTS = 20000
N_PAD = 20480
N_KEY = 2048
NSAMPLE = 32
RADIUS2 = 0.8 * 0.8
C_BEV = 256
H = 176
W = 200
BOUND = 4.0
PADV = 1.0e6
INVALID = 1.0e10


def _fps_body(x_ref, y_ref, z_ref, pts_ref, d0_ref, out_ref):
    rows = lax.broadcasted_iota(jnp.int32, (8, 2560), 0)
    cols = lax.broadcasted_iota(jnp.int32, (8, 2560), 1)
    lin = rows * 2560 + cols
    lanes = lax.broadcasted_iota(jnp.int32, (1, 128), 1)

    row0 = pts_ref[0:1, :]
    lx0 = row0[0:1, 0:1]
    ly0 = row0[0:1, 1:2]
    lz0 = row0[0:1, 2:3]
    out_ref[0:1, :] = jnp.where(
        lanes == 0, lx0, jnp.where(lanes == 1, ly0, jnp.where(lanes == 2, lz0, 0.0)))

    def body(i, carry):
        dists, lx, ly, lz = carry
        dx = x_ref[...] - lx
        dy = y_ref[...] - ly
        dz = z_ref[...] - lz
        d = dx * dx + dy * dy + dz * dz
        dn = jnp.minimum(dists, d)
        m = jnp.max(dn)
        sel = jnp.min(jnp.where(dn == m, lin, jnp.int32(2 ** 30)))
        prow = pts_ref[pl.ds(sel, 1), :]
        nlx = prow[0:1, 0:1]
        nly = prow[0:1, 1:2]
        nlz = prow[0:1, 2:3]
        row = jnp.where(
            lanes == 0, nlx, jnp.where(lanes == 1, nly, jnp.where(lanes == 2, nlz, 0.0)))
        out_ref[pl.ds(i, 1), :] = row
        return (dn, nlx, nly, nlz)

    lax.fori_loop(1, N_KEY, body, (d0_ref[...], lx0, ly0, lz0))


def _fps(x8, y8, z8, pts8, d0):
    return pl.pallas_call(
        _fps_body,
        out_shape=jax.ShapeDtypeStruct((N_KEY, 128), jnp.float32),
    )(x8, y8, z8, pts8, d0)


KB = 16


def _ballq_body(key_ref, x_ref, y_ref, z_ref, out_ref, m_ref):
    kx = key_ref[:, 0:1]
    ky = key_ref[:, 1:2]
    kz = key_ref[:, 2:3]
    dx = kx - x_ref[...]
    dy = ky - y_ref[...]
    dz = kz - z_ref[...]
    d2 = dx * dx + dy * dy + dz * dz
    m_ref[...] = jnp.where(d2 <= RADIUS2, d2, INVALID)

    lanelin = lax.broadcasted_iota(jnp.int32, (KB, N_PAD), 1)
    out_ref[...] = jnp.zeros((KB, 128), jnp.int32)

    first_sel = None
    for s in range(NSAMPLE):
        cur = m_ref[...]
        m = jnp.min(cur, axis=1, keepdims=True)
        eq = cur == m
        sel = jnp.min(jnp.where(eq, lanelin, jnp.int32(2 ** 30)),
                      axis=1, keepdims=True)
        m_ref[...] = jnp.where(eq, INVALID, cur)
        if s == 0:
            first_sel = sel
            rec = sel
        else:
            rec = jnp.where(m < 1.0e9, sel, first_sel)
        out_ref[:, s:s + 1] = rec


def _ballq(keyrows, x1, y1, z1):
    return pl.pallas_call(
        _ballq_body,
        grid=(N_KEY // KB,),
        in_specs=[
            pl.BlockSpec((KB, 128), lambda i: (i, 0)),
            pl.BlockSpec((1, N_PAD), lambda i: (0, 0)),
            pl.BlockSpec((1, N_PAD), lambda i: (0, 0)),
            pl.BlockSpec((1, N_PAD), lambda i: (0, 0)),
        ],
        out_specs=pl.BlockSpec((KB, 128), lambda i: (i, 0)),
        out_shape=jax.ShapeDtypeStruct((N_KEY, 128), jnp.int32),
        scratch_shapes=[pltpu.VMEM((KB, N_PAD), jnp.float32)],
    )(keyrows, x1, y1, z1)


NW = 32
NBR_TOTAL = N_KEY * NSAMPLE
NBR_PER_W = NBR_TOTAL // NW
NBR_CHUNKS = NBR_PER_W // 128


def _sc_nbr_gather(points128, idx3d):
    mesh = plsc.VectorSubcoreMesh(core_axis_name="c", subcore_axis_name="s")

    @functools.partial(
        pl.kernel, mesh=mesh,
        out_type=jax.ShapeDtypeStruct((NBR_TOTAL, 128), jnp.float32),
        scratch_types=[
            pltpu.VMEM((NBR_CHUNKS, 128), jnp.int32),
            pltpu.VMEM((128, 128), jnp.float32),
            pltpu.VMEM((128, 128), jnp.float32),
            pltpu.SemaphoreType.DMA,
            pltpu.SemaphoreType.DMA,
        ],
    )
    def k(table_hbm, idx_hbm, out_hbm, idx_v, buf0, buf1, sem0, sem1):
        wid = lax.axis_index("s") * 2 + lax.axis_index("c")
        base = wid * NBR_PER_W
        pltpu.sync_copy(idx_hbm.at[wid], idx_v)
        bufs = (buf0, buf1)
        sems = (sem0, sem1)
        copies = [None, None]
        for j in range(NBR_CHUNKS):
            p = j % 2
            if copies[p] is not None:
                copies[p].wait()
                pltpu.sync_copy(bufs[p],
                                out_hbm.at[pl.ds(base + (j - 2) * 128, 128)])
            copies[p] = pltpu.async_copy(
                table_hbm.at[idx_v.at[j]], bufs[p], sems[p])
        for j in (NBR_CHUNKS - 2, NBR_CHUNKS - 1):
            p = j % 2
            copies[p].wait()
            pltpu.sync_copy(bufs[p], out_hbm.at[pl.ds(base + j * 128, 128)])

    return k(points128, idx3d)


KEY_PER_W = N_KEY // NW


def _sc_bev_gather(table, kx, ky):
    mesh = plsc.VectorSubcoreMesh(core_axis_name="c", subcore_axis_name="s")
    out1 = jax.ShapeDtypeStruct((N_KEY, C_BEV), jnp.float32)

    @functools.partial(
        pl.kernel, mesh=mesh,
        out_type=(out1, out1, out1, out1),
        scratch_types=[
            pltpu.VMEM((KEY_PER_W,), jnp.float32),
            pltpu.VMEM((KEY_PER_W,), jnp.float32),
            pltpu.VMEM((KEY_PER_W,), jnp.int32),
            pltpu.VMEM((KEY_PER_W,), jnp.int32),
            pltpu.VMEM((KEY_PER_W,), jnp.int32),
            pltpu.VMEM((KEY_PER_W,), jnp.int32),
            pltpu.VMEM((KEY_PER_W, C_BEV), jnp.float32),
            pltpu.VMEM((KEY_PER_W, C_BEV), jnp.float32),
            pltpu.VMEM((KEY_PER_W, C_BEV), jnp.float32),
            pltpu.VMEM((KEY_PER_W, C_BEV), jnp.float32),
            pltpu.SemaphoreType.DMA,
        ],
    )
    def k(table_hbm, kx_hbm, ky_hbm, o00, o01, o10, o11,
          kx_v, ky_v, i00, i01, i10, i11, f00, f01, f10, f11, sem):
        wid = lax.axis_index("s") * 2 + lax.axis_index("c")
        base = wid * KEY_PER_W
        pltpu.sync_copy(kx_hbm.at[pl.ds(base, KEY_PER_W)], kx_v)
        pltpu.sync_copy(ky_hbm.at[pl.ds(base, KEY_PER_W)], ky_v)
        for j in range(KEY_PER_W // 16):
            sl = pl.ds(j * 16, 16)
            x = kx_v[sl]
            y = ky_v[sl]
            u = (x + BOUND) / (2.0 * BOUND) * (W - 1)
            v = (y + BOUND) / (2.0 * BOUND) * (H - 1)
            u = jnp.minimum(jnp.maximum(u, 0.0), W - 1.0)
            v = jnp.minimum(jnp.maximum(v, 0.0), H - 1.0)
            u0 = u.astype(jnp.int32)
            v0 = v.astype(jnp.int32)
            u1 = jnp.minimum(u0 + 1, W - 1)
            v1 = jnp.minimum(v0 + 1, H - 1)
            i00[sl] = v0 * W + u0
            i01[sl] = v0 * W + u1
            i10[sl] = v1 * W + u0
            i11[sl] = v1 * W + u1
        copies = [
            pltpu.async_copy(table_hbm.at[i00], f00, sem),
            pltpu.async_copy(table_hbm.at[i01], f01, sem),
            pltpu.async_copy(table_hbm.at[i10], f10, sem),
            pltpu.async_copy(table_hbm.at[i11], f11, sem),
        ]
        for c in copies:
            c.wait()
        dst = pl.ds(base, KEY_PER_W)
        pltpu.sync_copy(f00, o00.at[dst])
        pltpu.sync_copy(f01, o01.at[dst])
        pltpu.sync_copy(f10, o10.at[dst])
        pltpu.sync_copy(f11, o11.at[dst])

    return k(table, kx, ky)


MB = 256


def _mlp_body(g_ref, key_ref, w1_ref, b1_ref, w2_ref, b2_ref, out_ref):
    g4 = g_ref[:, 0:4]
    a = jnp.dot(g4, w1_ref[...], preferred_element_type=jnp.float32)
    kxyz = key_ref[:, 0:3]
    b = jnp.dot(kxyz, w1_ref[0:3, :], preferred_element_type=jnp.float32)
    a3 = a.reshape(MB, NSAMPLE, 32)
    h1 = jnp.maximum(a3 - b[:, None, :] + b1_ref[0:1, :][None, :, :], 0.0)
    h2 = jnp.dot(h1.reshape(MB * NSAMPLE, 32), w2_ref[...],
                 preferred_element_type=jnp.float32)
    h2 = jnp.maximum(h2 + b2_ref[0:1, :], 0.0)
    out_ref[...] = jnp.max(h2.reshape(MB, NSAMPLE, 64), axis=1)


def _mlp(gathered, keyrows, W1, b1, W2, b2):
    return pl.pallas_call(
        _mlp_body,
        grid=(N_KEY // MB,),
        in_specs=[
            pl.BlockSpec((MB * NSAMPLE, 128), lambda i: (i, 0)),
            pl.BlockSpec((MB, 128), lambda i: (i, 0)),
            pl.BlockSpec((4, 32), lambda i: (0, 0)),
            pl.BlockSpec((1, 32), lambda i: (0, 0)),
            pl.BlockSpec((32, 64), lambda i: (0, 0)),
            pl.BlockSpec((1, 64), lambda i: (0, 0)),
        ],
        out_specs=pl.BlockSpec((MB, 64), lambda i: (i, 0)),
        out_shape=jax.ShapeDtypeStruct((N_KEY, 64), jnp.float32),
    )(gathered, keyrows, W1, b1, W2, b2)


def _blend_body(pnet_ref, f00_ref, f01_ref, f10_ref, f11_ref, key_ref, out_ref):
    kx = key_ref[:, 0:1]
    ky = key_ref[:, 1:2]
    u = (kx + BOUND) / (2.0 * BOUND) * (W - 1)
    v = (ky + BOUND) / (2.0 * BOUND) * (H - 1)
    u = jnp.clip(u, 0.0, W - 1.0)
    v = jnp.clip(v, 0.0, H - 1.0)
    wu = u - jnp.floor(u)
    wv = v - jnp.floor(v)
    bev = (f00_ref[...] * (1.0 - wu) * (1.0 - wv)
           + f01_ref[...] * wu * (1.0 - wv)
           + f10_ref[...] * (1.0 - wu) * wv
           + f11_ref[...] * wu * wv)
    out_ref[:, 0:64] = pnet_ref[...]
    out_ref[:, 64:320] = bev


def _blend(pnet, f00, f01, f10, f11, keyrows):
    return pl.pallas_call(
        _blend_body,
        out_shape=jax.ShapeDtypeStruct((N_KEY, 320), jnp.float32),
    )(pnet, f00, f01, f10, f11, keyrows)


def kernel(points, bev_volume, W1, b1, W2, b2):
    xyz = points[:, :3]
    pad = jnp.full((N_PAD - N_POINTS,), PADV, jnp.float32)
    xp = jnp.concatenate([xyz[:, 0], pad])
    yp = jnp.concatenate([xyz[:, 1], pad])
    zp = jnp.concatenate([xyz[:, 2], pad])
    d0 = jnp.concatenate([
        jnp.full((N_POINTS,), INVALID, jnp.float32),
        jnp.full((N_PAD - N_POINTS,), -INVALID, jnp.float32),
    ]).reshape(8, 2560)

    pts8 = jnp.pad(xyz, ((0, N_PAD - N_POINTS), (0, 5)))
    keyrows = _fps(xp.reshape(8, 2560), yp.reshape(8, 2560),
                   zp.reshape(8, 2560), pts8, d0)

    nbr_pad = _ballq(keyrows, xp.reshape(1, N_PAD), yp.reshape(1, N_PAD),
                     zp.reshape(1, N_PAD))
    idx3d = nbr_pad[:, :NSAMPLE].reshape(NW, NBR_CHUNKS, 128)

    points128 = jnp.pad(points, ((0, 0), (0, 124)))
    gathered = _sc_nbr_gather(points128, idx3d)

    bev_t = jnp.transpose(bev_volume[0], (1, 2, 0)).reshape(H * W, C_BEV)
    kx = keyrows[:, 0]
    ky = keyrows[:, 1]
    f00, f01, f10, f11 = _sc_bev_gather(bev_t, kx, ky)

    pnet = _mlp(gathered, keyrows, W1, b1.reshape(1, 32), W2, b2.reshape(1, 64))

    return _blend(pnet, f00, f01, f10, f11, keyrows)
